# Initial kernel scaffold; baseline (speedup 1.0000x reference)
import jax
import jax.numpy as jnp
from jax.experimental import pallas as pl


def kernel(edge_index, emb, W_hg, b_hg, W_lin, b_lin):
    raise NotImplementedError("write your pallas kernel here")


# speedup vs baseline: 12.8135x; 12.8135x over previous
import functools

import jax
import jax.numpy as jnp
from jax import lax
from jax.experimental import pallas as pl
from jax.experimental.pallas import tpu as pltpu
from jax.experimental.pallas import tpu_sc as plsc

NC = 2
NS = 16
NW = NC * NS
PAD = 16


def _leaky(x):
    return jnp.where(x >= 0, x, 0.01 * x)


def _xt_body(emb_ref, w_ref, out_ref):
    c = emb_ref.shape[1]
    x = lax.dot_general(emb_ref[...], w_ref[...], (((1,), (1,)), ((), ())),
                        preferred_element_type=jnp.float32,
                        precision=lax.Precision.HIGHEST)
    out_ref[:, :c] = x
    col = lax.broadcasted_iota(jnp.int32, (emb_ref.shape[0], PAD), 1)
    out_ref[:, c:] = jnp.where(col == 0, 1.0, 0.0)


def _combine_body(p0_ref, p1_ref, out_ref):
    c = p0_ref.shape[1] - PAD
    s = p0_ref[...] + p1_ref[...]
    cnt = s[:, c:c + 1]
    inv = jnp.where(cnt > 0, 1.0 / cnt, 0.0)
    out_ref[:, :c] = s[:, :c] * inv
    col = lax.broadcasted_iota(jnp.int32, (s.shape[0], PAD), 1)
    out_ref[:, c:] = jnp.where(col == 0, 1.0, 0.0)


def _finish_body(nrows, p0_ref, p1_ref, bhg_ref, wlin_ref, blin_ref, out_ref,
                 acc_ref):
    c = p0_ref.shape[1] - PAD
    br = p0_ref.shape[0]
    i = pl.program_id(0)

    @pl.when(i == 0)
    def _():
        acc_ref[...] = jnp.zeros_like(acc_ref)

    s = p0_ref[...] + p1_ref[...]
    cnt = s[:, c:c + 1]
    inv = jnp.where(cnt > 0, 1.0 / cnt, 0.0)
    y = _leaky(s[:, :c] * inv + bhg_ref[...])
    row = i * br + lax.broadcasted_iota(jnp.int32, (br, c), 0)
    y = jnp.where(row < nrows, y, 0.0)
    acc_ref[...] += lax.dot_general(y, y, (((0,), (0,)), ((), ())),
                                    preferred_element_type=jnp.float32,
                                    precision=lax.Precision.HIGHEST)

    @pl.when(i == pl.num_programs(0) - 1)
    def _():
        z = lax.dot_general(acc_ref[...], wlin_ref[...], (((1,), (1,)), ((), ())),
                            preferred_element_type=jnp.float32,
                            precision=lax.Precision.HIGHEST) + blin_ref[...]
        out_ref[...] = _leaky(z)


def _sc_pass_body(npad, cp, k, chunks, ew,
                  gidx, sidx, table, zeros, out, gv, sv, rows, acc, sem):
    c = lax.axis_index("c")
    s = lax.axis_index("s")
    wid = s * NC + c
    rps = npad // NS

    pltpu.sync_copy(zeros.at[pl.ds(s * rps, rps)], acc.at[pl.ds(s * rps, rps)])
    plsc.subcore_barrier()

    base = wid * ew

    def body(i, carry):
        off = base + i * k
        pltpu.sync_copy(gidx.at[pl.ds(off, k)], gv)
        pltpu.sync_copy(sidx.at[pl.ds(off, k)], sv)
        pltpu.async_copy(table.at[gv], rows, sem).wait()
        pltpu.sync_copy(rows, acc.at[sv], add=True)
        return carry

    lax.fori_loop(0, chunks, body, 0)
    plsc.subcore_barrier()
    pltpu.sync_copy(acc.at[pl.ds(s * rps, rps)],
                    out.at[c, pl.ds(s * rps, rps)])


@functools.cache
def _make_sc_pass(npad, cp, e):
    ew = e // NW
    k = 80
    chunks = ew // k
    assert ew % k == 0 and npad % (8 * NS) == 0
    body = functools.partial(_sc_pass_body, npad, cp, k, chunks, ew)
    return pl.kernel(
        body,
        out_type=jax.ShapeDtypeStruct((NC, npad, cp), jnp.float32),
        mesh=plsc.VectorSubcoreMesh(core_axis_name="c", subcore_axis_name="s",
                                    num_cores=NC, num_subcores=NS),
        scratch_types=[
            pltpu.VMEM((k,), jnp.int32),
            pltpu.VMEM((k,), jnp.int32),
            pltpu.VMEM((k, cp), jnp.float32),
            pltpu.VMEM_SHARED((npad, cp), jnp.float32),
            pltpu.SemaphoreType.DMA,
        ],
        compiler_params=pltpu.CompilerParams(use_tc_tiling_on_sc=False),
    )


def kernel(edge_index, emb, W_hg, b_hg, W_lin, b_lin):
    n, c = emb.shape
    e = edge_index.shape[1]
    cp = c + PAD
    brp = 1024
    npad = -(-n // brp) * brp
    assert npad % (8 * NS) == 0
    br = 1000
    grid = n // br
    gridp = npad // brp

    node_idx = edge_index[0]
    hedge_idx = edge_index[1]
    zeros = jnp.zeros((npad, cp), jnp.float32)

    xt = pl.pallas_call(
        _xt_body,
        grid=(grid,),
        in_specs=[pl.BlockSpec((br, c), lambda i: (i, 0)),
                  pl.BlockSpec((c, c), lambda i: (0, 0))],
        out_specs=pl.BlockSpec((br, cp), lambda i: (i, 0)),
        out_shape=jax.ShapeDtypeStruct((n, cp), jnp.float32),
    )(emb, W_hg)

    sc_pass = _make_sc_pass(npad, cp, e)

    part1 = sc_pass(node_idx, hedge_idx, xt, zeros)

    ef = pl.pallas_call(
        _combine_body,
        grid=(gridp,),
        in_specs=[pl.BlockSpec((brp, cp), lambda i: (i, 0)),
                  pl.BlockSpec((brp, cp), lambda i: (i, 0))],
        out_specs=pl.BlockSpec((brp, cp), lambda i: (i, 0)),
        out_shape=jax.ShapeDtypeStruct((npad, cp), jnp.float32),
    )(part1[0], part1[1])

    part2 = sc_pass(hedge_idx, node_idx, ef, zeros)

    out = pl.pallas_call(
        functools.partial(_finish_body, n),
        grid=(gridp,),
        in_specs=[pl.BlockSpec((brp, cp), lambda i: (i, 0)),
                  pl.BlockSpec((brp, cp), lambda i: (i, 0)),
                  pl.BlockSpec((1, c), lambda i: (0, 0)),
                  pl.BlockSpec((c, c), lambda i: (0, 0)),
                  pl.BlockSpec((1, c), lambda i: (0, 0))],
        out_specs=pl.BlockSpec((c, c), lambda i: (0, 0)),
        out_shape=jax.ShapeDtypeStruct((c, c), jnp.float32),
        scratch_shapes=[pltpu.VMEM((c, c), jnp.float32)],
    )(part2[0], part2[1], b_hg.reshape(1, c), W_lin, b_lin.reshape(1, c))

    return out
